# baseline (speedup 1.0000x reference)
import functools

import jax
import jax.numpy as jnp
from jax import lax
from jax.experimental import pallas as pl
from jax.experimental.pallas import tpu as pltpu
from jax.experimental.pallas import tpu_sc as plsc

N = 10000
NPAD = 10240
E = 320000
F = 128
N_CLASSES = 40

NC = 2
NS = 16
NW = NC * NS
EPT = E // NW
EPTP = 10240
BATCH = 80
NBATCH = EPTP // BATCH
CHUNK = 64
NCHUNK = NBATCH // CHUNK
DEPTH = 2
BATCH_G = 128
NBATCH_G = EPTP // BATCH_G
DEPTH_G = 4
QUADS_G = NBATCH_G // DEPTH_G

RPT = NPAD // NS

_MESH = plsc.VectorSubcoreMesh(core_axis_name="c", subcore_axis_name="s")

ROWB = 1024
NBLK = NPAD // ROWB


@functools.partial(
    pl.kernel,
    out_type=jax.ShapeDtypeStruct((NC, NPAD, F), jnp.float32),
    mesh=_MESH,
    scratch_types=[
        pltpu.VMEM((NBATCH_G, BATCH_G), jnp.int32),
        pltpu.VMEM((BATCH_G, F), jnp.float32),
        pltpu.VMEM_SHARED((NPAD, F), jnp.float32),
        pltpu.SemaphoreType.DMA,
        pltpu.SemaphoreType.DMA,
        pltpu.SemaphoreType.DMA,
        pltpu.SemaphoreType.DMA,
    ],
)
def _deg_kernel(dst_hbm, zeros_hbm, ones_hbm, out_hbm, dst_v, ones_v, cnt_sh,
                ssem0, ssem1, ssem2, ssem3):
    ssem = (ssem0, ssem1, ssem2, ssem3)
    c = lax.axis_index("c")
    s = lax.axis_index("s")
    wid = c * NS + s
    pltpu.sync_copy(zeros_hbm.at[pl.ds(s * RPT, RPT)],
                    cnt_sh.at[pl.ds(s * RPT, RPT)])
    pltpu.sync_copy(ones_hbm, ones_v)
    pltpu.sync_copy(dst_hbm.at[wid], dst_v)
    plsc.subcore_barrier()

    def sstart(j, k):
        pltpu.make_async_copy(ones_v, cnt_sh.at[dst_v.at[j]],
                              ssem[k]).start(add=True)

    def swait(j, k):
        pltpu.make_async_copy(ones_v, cnt_sh.at[dst_v.at[j]],
                              ssem[k]).wait()

    for k in range(DEPTH_G):
        sstart(k, k)

    def body(i, carry):
        j0 = i * DEPTH_G
        for k in range(DEPTH_G):
            swait(j0 + k, k)
            sstart(j0 + DEPTH_G + k, k)
        return carry

    lax.fori_loop(0, QUADS_G - 1, body, 0)
    for k in range(DEPTH_G):
        swait((QUADS_G - 1) * DEPTH_G + k, k)
    plsc.subcore_barrier()
    pltpu.sync_copy(cnt_sh.at[pl.ds(s * RPT, RPT)],
                    out_hbm.at[c, pl.ds(s * RPT, RPT)])


@functools.partial(
    pl.kernel,
    out_type=jax.ShapeDtypeStruct((NC, NPAD, F), jnp.float32),
    mesh=_MESH,
    scratch_types=[
        pltpu.VMEM((CHUNK, BATCH), jnp.int32),
        pltpu.VMEM((CHUNK, BATCH), jnp.int32),
        pltpu.VMEM((DEPTH, BATCH, F), jnp.float32),
        pltpu.VMEM_SHARED((NPAD, F), jnp.float32),
        pltpu.SemaphoreType.DMA,
        pltpu.SemaphoreType.DMA,
        pltpu.SemaphoreType.DMA,
        pltpu.SemaphoreType.DMA,
    ],
)
def _ppass_kernel(src_hbm, dst_hbm, x_hbm, zeros_hbm, out_hbm,
                  src_v, dst_v, rows_v, agg_sh, gsem0, gsem1, ssem0, ssem1):
    gsem = (gsem0, gsem1)
    ssem = (ssem0, ssem1)
    c = lax.axis_index("c")
    s = lax.axis_index("s")
    wid = c * NS + s
    pltpu.sync_copy(zeros_hbm.at[pl.ds(s * RPT, RPT)],
                    agg_sh.at[pl.ds(s * RPT, RPT)])
    plsc.subcore_barrier()

    def gstart(j, k):
        pltpu.make_async_copy(x_hbm.at[src_v.at[j]], rows_v.at[k],
                              gsem[k]).start()

    def gwait(j, k):
        pltpu.make_async_copy(x_hbm.at[src_v.at[j]], rows_v.at[k],
                              gsem[k]).wait()

    def sstart(j, k):
        pltpu.make_async_copy(rows_v.at[k], agg_sh.at[dst_v.at[j]],
                              ssem[k]).start(add=True)

    def swait(j, k):
        pltpu.make_async_copy(rows_v.at[k], agg_sh.at[dst_v.at[j]],
                              ssem[k]).wait()

    for chunk in range(NCHUNK):
        pltpu.sync_copy(src_hbm.at[wid, pl.ds(chunk * CHUNK, CHUNK)], src_v)
        pltpu.sync_copy(dst_hbm.at[wid, pl.ds(chunk * CHUNK, CHUNK)], dst_v)
        for k in range(DEPTH):
            gstart(k, k)

        def body(i, carry):
            j0 = i * DEPTH
            for k in range(DEPTH):
                gwait(j0 + k, k)
                sstart(j0 + k, k)
            for k in range(DEPTH):
                swait(j0 + k, k)
                gstart(j0 + DEPTH + k, k)
            return carry

        lax.fori_loop(0, CHUNK // DEPTH - 1, body, 0)
        j0 = CHUNK - DEPTH
        for k in range(DEPTH):
            gwait(j0 + k, k)
            sstart(j0 + k, k)
        for k in range(DEPTH):
            swait(j0 + k, k)

    plsc.subcore_barrier()
    pltpu.sync_copy(agg_sh.at[pl.ds(s * RPT, RPT)],
                    out_hbm.at[c, pl.ds(s * RPT, RPT)])


def _prep_body(deg_ref, x_ref, dinv_ref, dinv2_ref, xs_ref):
    d = jnp.maximum(deg_ref[0][:, :16] + deg_ref[1][:, :16], 1.0)
    dinv2 = 1.0 / d
    dinv = jnp.sqrt(dinv2)
    dinv_ref[...] = dinv
    dinv2_ref[...] = dinv2
    xs_ref[...] = x_ref[...] * dinv[:, :1]


def _prep(deg, x):
    return pl.pallas_call(
        _prep_body,
        grid=(NBLK,),
        in_specs=[
            pl.BlockSpec((NC, ROWB, F), lambda i: (0, i, 0)),
            pl.BlockSpec((ROWB, F), lambda i: (i, 0)),
        ],
        out_specs=[
            pl.BlockSpec((ROWB, 16), lambda i: (i, 0)),
            pl.BlockSpec((ROWB, 16), lambda i: (i, 0)),
            pl.BlockSpec((ROWB, F), lambda i: (i, 0)),
        ],
        out_shape=[
            jax.ShapeDtypeStruct((NPAD, 16), jnp.float32),
            jax.ShapeDtypeStruct((NPAD, 16), jnp.float32),
            jax.ShapeDtypeStruct((NPAD, F), jnp.float32),
        ],
    )(deg, x)


def _mid_body(a1_ref, dinv2_ref, t1s_ref):
    t1s_ref[...] = (a1_ref[0] + a1_ref[1]) * (-dinv2_ref[:, :1])


def _mid(a1, dinv2):
    return pl.pallas_call(
        _mid_body,
        grid=(NBLK,),
        in_specs=[
            pl.BlockSpec((NC, ROWB, F), lambda i: (0, i, 0)),
            pl.BlockSpec((ROWB, 16), lambda i: (i, 0)),
        ],
        out_specs=pl.BlockSpec((ROWB, F), lambda i: (i, 0)),
        out_shape=jax.ShapeDtypeStruct((NPAD, F), jnp.float32),
    )(a1, dinv2)


def _layer_body(x_ref, a1_ref, a2_ref, dinv_ref, w1_ref, w2_ref, w3_ref,
                b_ref, h_ref, hs_ref):
    dinv = dinv_ref[:, :1]
    u1 = (a1_ref[0] + a1_ref[1]) * dinv
    u2 = (a2_ref[0] + a2_ref[1]) * dinv
    h = jnp.dot(x_ref[...], w1_ref[...], preferred_element_type=jnp.float32)
    h += jnp.dot(u1, w2_ref[...], preferred_element_type=jnp.float32)
    h += jnp.dot(u2, w3_ref[...], preferred_element_type=jnp.float32)
    h += b_ref[...]
    row = (pl.program_id(0) * ROWB
           + lax.broadcasted_iota(jnp.int32, (ROWB, 1), 0))
    h = jnp.where(row < N, h, 0.0)
    h_ref[...] = h
    hs_ref[...] = h * dinv


def _layer(x, a1, a2, dinv, w1, w2, w3, b):
    return pl.pallas_call(
        _layer_body,
        grid=(NBLK,),
        in_specs=[
            pl.BlockSpec((ROWB, F), lambda i: (i, 0)),
            pl.BlockSpec((NC, ROWB, F), lambda i: (0, i, 0)),
            pl.BlockSpec((NC, ROWB, F), lambda i: (0, i, 0)),
            pl.BlockSpec((ROWB, 16), lambda i: (i, 0)),
            pl.BlockSpec((F, F), lambda i: (0, 0)),
            pl.BlockSpec((F, F), lambda i: (0, 0)),
            pl.BlockSpec((F, F), lambda i: (0, 0)),
            pl.BlockSpec((1, F), lambda i: (0, 0)),
        ],
        out_specs=[
            pl.BlockSpec((ROWB, F), lambda i: (i, 0)),
            pl.BlockSpec((ROWB, F), lambda i: (i, 0)),
        ],
        out_shape=[
            jax.ShapeDtypeStruct((NPAD, F), jnp.float32),
            jax.ShapeDtypeStruct((NPAD, F), jnp.float32),
        ],
    )(x, a1, a2, dinv, w1, w2, w3, b)


def _final_body(x_ref, a1_ref, a2_ref, dinv_ref, w1_ref, w2_ref, w3_ref,
                b_ref, out_ref):
    i = pl.program_id(0)

    @pl.when(i == 0)
    def _():
        out_ref[...] = jnp.zeros_like(out_ref)

    dinv = dinv_ref[:, :1]
    u1 = (a1_ref[0] + a1_ref[1]) * dinv
    u2 = (a2_ref[0] + a2_ref[1]) * dinv
    h = jnp.dot(x_ref[...], w1_ref[...], preferred_element_type=jnp.float32)
    h += jnp.dot(u1, w2_ref[...], preferred_element_type=jnp.float32)
    h += jnp.dot(u2, w3_ref[...], preferred_element_type=jnp.float32)
    row = (i * ROWB + lax.broadcasted_iota(jnp.int32, (ROWB, 1), 0))
    h = jnp.where(row < N, h, 0.0)
    out_ref[...] += jnp.sum(h, axis=0, keepdims=True)

    @pl.when(i == NBLK - 1)
    def _():
        out_ref[...] = out_ref[...] * (1.0 / N) + b_ref[...]


def _final(x, a1, a2, dinv, w1, w2, w3, b):
    return pl.pallas_call(
        _final_body,
        grid=(NBLK,),
        in_specs=[
            pl.BlockSpec((ROWB, F), lambda i: (i, 0)),
            pl.BlockSpec((NC, ROWB, F), lambda i: (0, i, 0)),
            pl.BlockSpec((NC, ROWB, F), lambda i: (0, i, 0)),
            pl.BlockSpec((ROWB, 16), lambda i: (i, 0)),
            pl.BlockSpec((F, N_CLASSES), lambda i: (0, 0)),
            pl.BlockSpec((F, N_CLASSES), lambda i: (0, 0)),
            pl.BlockSpec((F, N_CLASSES), lambda i: (0, 0)),
            pl.BlockSpec((1, N_CLASSES), lambda i: (0, 0)),
        ],
        out_specs=pl.BlockSpec((1, N_CLASSES), lambda i: (0, 0)),
        out_shape=jax.ShapeDtypeStruct((1, N_CLASSES), jnp.float32),
    )(x, a1, a2, dinv, w1, w2, w3, b)


def kernel(features, edge_index, W0, b0, W1, b1, W2, b2):
    pad = ((0, 0), (0, EPTP - EPT))
    src2 = jnp.pad(edge_index[0].reshape(NW, EPT), pad,
                   constant_values=NPAD - 1)
    dst2 = jnp.pad(edge_index[1].reshape(NW, EPT), pad,
                   constant_values=NPAD - 1)
    src = src2.reshape(NW, NBATCH, BATCH)
    dst = dst2.reshape(NW, NBATCH, BATCH)
    dst_g = dst2.reshape(NW, NBATCH_G, BATCH_G)
    zeros_f = jnp.zeros((NPAD, F), jnp.float32)
    ones_f = jnp.ones((BATCH_G, F), jnp.float32)
    feats = jnp.pad(features, ((0, NPAD - N), (0, 0)))

    deg = _deg_kernel(dst_g, zeros_f, ones_f)
    dinv, dinv2, xs = _prep(deg, feats)

    def cheb_weights(w, width):
        wa, wb, wc = w[:width], w[width:2 * width], w[2 * width:]
        return wa - wc, -wb, -2.0 * wc

    x = feats
    for w, b in ((W0, b0), (W1, b1)):
        w1e, w2e, w3e = cheb_weights(w, F)
        a1 = _ppass_kernel(src, dst, xs, zeros_f)
        t1s = _mid(a1, dinv2)
        a2 = _ppass_kernel(src, dst, t1s, zeros_f)
        x, xs = _layer(x, a1, a2, dinv, w1e, w2e, w3e, b.reshape(1, F))

    w1e, w2e, w3e = cheb_weights(W2, F)
    a1 = _ppass_kernel(src, dst, xs, zeros_f)
    t1s = _mid(a1, dinv2)
    a2 = _ppass_kernel(src, dst, t1s, zeros_f)
    return _final(x, a1, a2, dinv, w1e, w2e, w3e, b2.reshape(1, N_CLASSES))


# speedup vs baseline: 1.9681x; 1.9681x over previous
import functools

import jax
import jax.numpy as jnp
from jax import lax
from jax.experimental import pallas as pl
from jax.experimental.pallas import tpu as pltpu
from jax.experimental.pallas import tpu_sc as plsc

N = 10000
NPAD = 10240
E = 320000
F = 128
N_CLASSES = 40

NC = 2
NS = 16
NW = NC * NS
EPT = E // NW
BATCH = 80
NBATCH = EPT // BATCH

RPT = NPAD // NS

_MESH = plsc.VectorSubcoreMesh(core_axis_name="c", subcore_axis_name="s")

ROWB = 1024
NBLK = NPAD // ROWB


@functools.partial(
    pl.kernel,
    out_type=jax.ShapeDtypeStruct((NC, NPAD, F), jnp.float32),
    mesh=_MESH,
    scratch_types=[
        pltpu.VMEM((NBATCH, BATCH), jnp.int32),
        pltpu.VMEM((BATCH, F), jnp.float32),
        pltpu.VMEM_SHARED((NPAD, F), jnp.float32),
    ],
)
def _deg_kernel(dst_hbm, zeros_hbm, ones_hbm, out_hbm, dst_v, ones_v, cnt_sh):
    c = lax.axis_index("c")
    s = lax.axis_index("s")
    wid = c * NS + s
    pltpu.sync_copy(zeros_hbm.at[pl.ds(s * RPT, RPT)],
                    cnt_sh.at[pl.ds(s * RPT, RPT)])
    pltpu.sync_copy(ones_hbm, ones_v)
    pltpu.sync_copy(dst_hbm.at[wid], dst_v)
    plsc.subcore_barrier()

    def body(j, carry):
        pltpu.sync_copy(ones_v, cnt_sh.at[dst_v.at[j]], add=True)
        return carry

    lax.fori_loop(0, NBATCH, body, 0)
    plsc.subcore_barrier()
    pltpu.sync_copy(cnt_sh.at[pl.ds(s * RPT, RPT)],
                    out_hbm.at[c, pl.ds(s * RPT, RPT)])


@functools.partial(
    pl.kernel,
    out_type=jax.ShapeDtypeStruct((NC, NPAD, F), jnp.float32),
    mesh=_MESH,
    scratch_types=[
        pltpu.VMEM((NBATCH, BATCH), jnp.int32),
        pltpu.VMEM((NBATCH, BATCH), jnp.int32),
        pltpu.VMEM((BATCH, F), jnp.float32),
        pltpu.VMEM_SHARED((NPAD, F), jnp.float32),
        pltpu.SemaphoreType.DMA,
    ],
)
def _ppass_kernel(src_hbm, dst_hbm, x_hbm, zeros_hbm, out_hbm,
                  src_v, dst_v, rows_v, agg_sh, sem):
    c = lax.axis_index("c")
    s = lax.axis_index("s")
    wid = c * NS + s
    pltpu.sync_copy(zeros_hbm.at[pl.ds(s * RPT, RPT)],
                    agg_sh.at[pl.ds(s * RPT, RPT)])
    pltpu.sync_copy(src_hbm.at[wid], src_v)
    pltpu.sync_copy(dst_hbm.at[wid], dst_v)
    plsc.subcore_barrier()

    def body(j, carry):
        pltpu.async_copy(x_hbm.at[src_v.at[j]], rows_v, sem).wait()
        pltpu.sync_copy(rows_v, agg_sh.at[dst_v.at[j]], add=True)
        return carry

    lax.fori_loop(0, NBATCH, body, 0)
    plsc.subcore_barrier()
    pltpu.sync_copy(agg_sh.at[pl.ds(s * RPT, RPT)],
                    out_hbm.at[c, pl.ds(s * RPT, RPT)])


def _prep_body(deg_ref, x_ref, dinv_ref, dinv2_ref, xs_ref):
    d = jnp.maximum(deg_ref[0][:, :16] + deg_ref[1][:, :16], 1.0)
    dinv2 = 1.0 / d
    dinv = jnp.sqrt(dinv2)
    dinv_ref[...] = dinv
    dinv2_ref[...] = dinv2
    xs_ref[...] = x_ref[...] * dinv[:, :1]


def _prep(deg, x):
    return pl.pallas_call(
        _prep_body,
        grid=(NBLK,),
        in_specs=[
            pl.BlockSpec((NC, ROWB, F), lambda i: (0, i, 0)),
            pl.BlockSpec((ROWB, F), lambda i: (i, 0)),
        ],
        out_specs=[
            pl.BlockSpec((ROWB, 16), lambda i: (i, 0)),
            pl.BlockSpec((ROWB, 16), lambda i: (i, 0)),
            pl.BlockSpec((ROWB, F), lambda i: (i, 0)),
        ],
        out_shape=[
            jax.ShapeDtypeStruct((NPAD, 16), jnp.float32),
            jax.ShapeDtypeStruct((NPAD, 16), jnp.float32),
            jax.ShapeDtypeStruct((NPAD, F), jnp.float32),
        ],
    )(deg, x)


def _mid_body(a1_ref, dinv2_ref, t1s_ref):
    t1s_ref[...] = (a1_ref[0] + a1_ref[1]) * (-dinv2_ref[:, :1])


def _mid(a1, dinv2):
    return pl.pallas_call(
        _mid_body,
        grid=(NBLK,),
        in_specs=[
            pl.BlockSpec((NC, ROWB, F), lambda i: (0, i, 0)),
            pl.BlockSpec((ROWB, 16), lambda i: (i, 0)),
        ],
        out_specs=pl.BlockSpec((ROWB, F), lambda i: (i, 0)),
        out_shape=jax.ShapeDtypeStruct((NPAD, F), jnp.float32),
    )(a1, dinv2)


def _layer_body(x_ref, a1_ref, a2_ref, dinv_ref, w1_ref, w2_ref, w3_ref,
                b_ref, h_ref, hs_ref):
    dinv = dinv_ref[:, :1]
    u1 = (a1_ref[0] + a1_ref[1]) * dinv
    u2 = (a2_ref[0] + a2_ref[1]) * dinv
    h = jnp.dot(x_ref[...], w1_ref[...], preferred_element_type=jnp.float32)
    h += jnp.dot(u1, w2_ref[...], preferred_element_type=jnp.float32)
    h += jnp.dot(u2, w3_ref[...], preferred_element_type=jnp.float32)
    h += b_ref[...]
    row = (pl.program_id(0) * ROWB
           + lax.broadcasted_iota(jnp.int32, (ROWB, 1), 0))
    h = jnp.where(row < N, h, 0.0)
    h_ref[...] = h
    hs_ref[...] = h * dinv


def _layer(x, a1, a2, dinv, w1, w2, w3, b):
    return pl.pallas_call(
        _layer_body,
        grid=(NBLK,),
        in_specs=[
            pl.BlockSpec((ROWB, F), lambda i: (i, 0)),
            pl.BlockSpec((NC, ROWB, F), lambda i: (0, i, 0)),
            pl.BlockSpec((NC, ROWB, F), lambda i: (0, i, 0)),
            pl.BlockSpec((ROWB, 16), lambda i: (i, 0)),
            pl.BlockSpec((F, F), lambda i: (0, 0)),
            pl.BlockSpec((F, F), lambda i: (0, 0)),
            pl.BlockSpec((F, F), lambda i: (0, 0)),
            pl.BlockSpec((1, F), lambda i: (0, 0)),
        ],
        out_specs=[
            pl.BlockSpec((ROWB, F), lambda i: (i, 0)),
            pl.BlockSpec((ROWB, F), lambda i: (i, 0)),
        ],
        out_shape=[
            jax.ShapeDtypeStruct((NPAD, F), jnp.float32),
            jax.ShapeDtypeStruct((NPAD, F), jnp.float32),
        ],
    )(x, a1, a2, dinv, w1, w2, w3, b)


def _final_body(x_ref, a1_ref, a2_ref, dinv_ref, w1_ref, w2_ref, w3_ref,
                b_ref, out_ref):
    i = pl.program_id(0)

    @pl.when(i == 0)
    def _():
        out_ref[...] = jnp.zeros_like(out_ref)

    dinv = dinv_ref[:, :1]
    u1 = (a1_ref[0] + a1_ref[1]) * dinv
    u2 = (a2_ref[0] + a2_ref[1]) * dinv
    h = jnp.dot(x_ref[...], w1_ref[...], preferred_element_type=jnp.float32)
    h += jnp.dot(u1, w2_ref[...], preferred_element_type=jnp.float32)
    h += jnp.dot(u2, w3_ref[...], preferred_element_type=jnp.float32)
    row = (i * ROWB + lax.broadcasted_iota(jnp.int32, (ROWB, 1), 0))
    h = jnp.where(row < N, h, 0.0)
    out_ref[...] += jnp.sum(h, axis=0, keepdims=True)

    @pl.when(i == NBLK - 1)
    def _():
        out_ref[...] = out_ref[...] * (1.0 / N) + b_ref[...]


def _final(x, a1, a2, dinv, w1, w2, w3, b):
    return pl.pallas_call(
        _final_body,
        grid=(NBLK,),
        in_specs=[
            pl.BlockSpec((ROWB, F), lambda i: (i, 0)),
            pl.BlockSpec((NC, ROWB, F), lambda i: (0, i, 0)),
            pl.BlockSpec((NC, ROWB, F), lambda i: (0, i, 0)),
            pl.BlockSpec((ROWB, 16), lambda i: (i, 0)),
            pl.BlockSpec((F, N_CLASSES), lambda i: (0, 0)),
            pl.BlockSpec((F, N_CLASSES), lambda i: (0, 0)),
            pl.BlockSpec((F, N_CLASSES), lambda i: (0, 0)),
            pl.BlockSpec((1, N_CLASSES), lambda i: (0, 0)),
        ],
        out_specs=pl.BlockSpec((1, N_CLASSES), lambda i: (0, 0)),
        out_shape=jax.ShapeDtypeStruct((1, N_CLASSES), jnp.float32),
    )(x, a1, a2, dinv, w1, w2, w3, b)


def kernel(features, edge_index, W0, b0, W1, b1, W2, b2):
    src = edge_index[0].reshape(NW, NBATCH, BATCH)
    dst = edge_index[1].reshape(NW, NBATCH, BATCH)
    zeros_f = jnp.zeros((NPAD, F), jnp.float32)
    ones_f = jnp.ones((BATCH, F), jnp.float32)
    feats = jnp.pad(features, ((0, NPAD - N), (0, 0)))

    deg = _deg_kernel(dst, zeros_f, ones_f)
    dinv, dinv2, xs = _prep(deg, feats)

    def cheb_weights(w, width):
        wa, wb, wc = w[:width], w[width:2 * width], w[2 * width:]
        return wa - wc, -wb, -2.0 * wc

    x = feats
    for w, b in ((W0, b0), (W1, b1)):
        w1e, w2e, w3e = cheb_weights(w, F)
        a1 = _ppass_kernel(src, dst, xs, zeros_f)
        t1s = _mid(a1, dinv2)
        a2 = _ppass_kernel(src, dst, t1s, zeros_f)
        x, xs = _layer(x, a1, a2, dinv, w1e, w2e, w3e, b.reshape(1, F))

    w1e, w2e, w3e = cheb_weights(W2, F)
    a1 = _ppass_kernel(src, dst, xs, zeros_f)
    t1s = _mid(a1, dinv2)
    a2 = _ppass_kernel(src, dst, t1s, zeros_f)
    return _final(x, a1, a2, dinv, w1e, w2e, w3e, b2.reshape(1, N_CLASSES))


# speedup vs baseline: 2.2303x; 1.1332x over previous
import functools

import jax
import jax.numpy as jnp
from jax import lax
from jax.experimental import pallas as pl
from jax.experimental.pallas import tpu as pltpu
from jax.experimental.pallas import tpu_sc as plsc

N = 10000
NPAD = 10240
E = 320000
F = 128
N_CLASSES = 40

NC = 2
NS = 16
NW = NC * NS
EPT = E // NW
EPTP = 10240
BATCH = 80
NBATCH = EPTP // BATCH
CHUNK = 64
NCHUNK = NBATCH // CHUNK
DEPTH = 2

RPT = NPAD // NS

_MESH = plsc.VectorSubcoreMesh(core_axis_name="c", subcore_axis_name="s")

ROWB = 1024
NBLK = NPAD // ROWB


@functools.partial(
    pl.kernel,
    out_type=jax.ShapeDtypeStruct((NC, NPAD, F), jnp.float32),
    mesh=_MESH,
    scratch_types=[
        pltpu.VMEM((NBATCH, BATCH), jnp.int32),
        pltpu.VMEM((BATCH, F), jnp.float32),
        pltpu.VMEM_SHARED((NPAD, F), jnp.float32),
    ],
)
def _deg_kernel(dst_hbm, zeros_hbm, ones_hbm, out_hbm, dst_v, ones_v, cnt_sh):
    c = lax.axis_index("c")
    s = lax.axis_index("s")
    wid = c * NS + s
    pltpu.sync_copy(zeros_hbm.at[pl.ds(s * RPT, RPT)],
                    cnt_sh.at[pl.ds(s * RPT, RPT)])
    pltpu.sync_copy(ones_hbm, ones_v)
    pltpu.sync_copy(dst_hbm.at[wid], dst_v)
    plsc.subcore_barrier()

    def body(j, carry):
        pltpu.sync_copy(ones_v, cnt_sh.at[dst_v.at[j]], add=True)
        return carry

    lax.fori_loop(0, NBATCH, body, 0)
    plsc.subcore_barrier()
    pltpu.sync_copy(cnt_sh.at[pl.ds(s * RPT, RPT)],
                    out_hbm.at[c, pl.ds(s * RPT, RPT)])


@functools.partial(
    pl.kernel,
    out_type=jax.ShapeDtypeStruct((NC, NPAD, F), jnp.float32),
    mesh=_MESH,
    scratch_types=[
        pltpu.VMEM((CHUNK, BATCH), jnp.int32),
        pltpu.VMEM((CHUNK, BATCH), jnp.int32),
        pltpu.VMEM((DEPTH, BATCH, F), jnp.float32),
        pltpu.VMEM_SHARED((NPAD, F), jnp.float32),
        pltpu.SemaphoreType.DMA,
        pltpu.SemaphoreType.DMA,
        pltpu.SemaphoreType.DMA,
        pltpu.SemaphoreType.DMA,
    ],
)
def _ppass_kernel(src_hbm, dst_hbm, x_hbm, zeros_hbm, out_hbm,
                  src_v, dst_v, rows_v, agg_sh, gsem0, gsem1, ssem0, ssem1):
    gsem = (gsem0, gsem1)
    ssem = (ssem0, ssem1)
    c = lax.axis_index("c")
    s = lax.axis_index("s")
    wid = c * NS + s
    pltpu.sync_copy(zeros_hbm.at[pl.ds(s * RPT, RPT)],
                    agg_sh.at[pl.ds(s * RPT, RPT)])
    plsc.subcore_barrier()

    def gstart(j, k):
        pltpu.make_async_copy(x_hbm.at[src_v.at[j]], rows_v.at[k],
                              gsem[k]).start()

    def gwait(j, k):
        pltpu.make_async_copy(x_hbm.at[src_v.at[j]], rows_v.at[k],
                              gsem[k]).wait()

    def sstart(j, k):
        pltpu.make_async_copy(rows_v.at[k], agg_sh.at[dst_v.at[j]],
                              ssem[k]).start(add=True)

    def swait(j, k):
        pltpu.make_async_copy(rows_v.at[k], agg_sh.at[dst_v.at[j]],
                              ssem[k]).wait()

    for chunk in range(NCHUNK):
        pltpu.sync_copy(src_hbm.at[wid, pl.ds(chunk * CHUNK, CHUNK)], src_v)
        pltpu.sync_copy(dst_hbm.at[wid, pl.ds(chunk * CHUNK, CHUNK)], dst_v)
        for k in range(DEPTH):
            gstart(k, k)

        def body(i, carry):
            j0 = i * DEPTH
            for k in range(DEPTH):
                gwait(j0 + k, k)
                sstart(j0 + k, k)
            for k in range(DEPTH):
                swait(j0 + k, k)
                gstart(j0 + DEPTH + k, k)
            return carry

        lax.fori_loop(0, CHUNK // DEPTH - 1, body, 0)
        j0 = CHUNK - DEPTH
        for k in range(DEPTH):
            gwait(j0 + k, k)
            sstart(j0 + k, k)
        for k in range(DEPTH):
            swait(j0 + k, k)

    plsc.subcore_barrier()
    pltpu.sync_copy(agg_sh.at[pl.ds(s * RPT, RPT)],
                    out_hbm.at[c, pl.ds(s * RPT, RPT)])


def _prep_body(deg_ref, x_ref, dinv_ref, dinv2_ref, xs_ref):
    d = jnp.maximum(deg_ref[0][:, :16] + deg_ref[1][:, :16], 1.0)
    dinv2 = 1.0 / d
    dinv = jnp.sqrt(dinv2)
    dinv_ref[...] = dinv
    dinv2_ref[...] = dinv2
    xs_ref[...] = x_ref[...] * dinv[:, :1]


def _prep(deg, x):
    return pl.pallas_call(
        _prep_body,
        grid=(NBLK,),
        in_specs=[
            pl.BlockSpec((NC, ROWB, F), lambda i: (0, i, 0)),
            pl.BlockSpec((ROWB, F), lambda i: (i, 0)),
        ],
        out_specs=[
            pl.BlockSpec((ROWB, 16), lambda i: (i, 0)),
            pl.BlockSpec((ROWB, 16), lambda i: (i, 0)),
            pl.BlockSpec((ROWB, F), lambda i: (i, 0)),
        ],
        out_shape=[
            jax.ShapeDtypeStruct((NPAD, 16), jnp.float32),
            jax.ShapeDtypeStruct((NPAD, 16), jnp.float32),
            jax.ShapeDtypeStruct((NPAD, F), jnp.float32),
        ],
    )(deg, x)


def _mid_body(a1_ref, dinv2_ref, t1s_ref):
    t1s_ref[...] = (a1_ref[0] + a1_ref[1]) * (-dinv2_ref[:, :1])


def _mid(a1, dinv2):
    return pl.pallas_call(
        _mid_body,
        grid=(NBLK,),
        in_specs=[
            pl.BlockSpec((NC, ROWB, F), lambda i: (0, i, 0)),
            pl.BlockSpec((ROWB, 16), lambda i: (i, 0)),
        ],
        out_specs=pl.BlockSpec((ROWB, F), lambda i: (i, 0)),
        out_shape=jax.ShapeDtypeStruct((NPAD, F), jnp.float32),
    )(a1, dinv2)


def _layer_body(x_ref, a1_ref, a2_ref, dinv_ref, w1_ref, w2_ref, w3_ref,
                b_ref, h_ref, hs_ref):
    dinv = dinv_ref[:, :1]
    u1 = (a1_ref[0] + a1_ref[1]) * dinv
    u2 = (a2_ref[0] + a2_ref[1]) * dinv
    h = jnp.dot(x_ref[...], w1_ref[...], preferred_element_type=jnp.float32)
    h += jnp.dot(u1, w2_ref[...], preferred_element_type=jnp.float32)
    h += jnp.dot(u2, w3_ref[...], preferred_element_type=jnp.float32)
    h += b_ref[...]
    row = (pl.program_id(0) * ROWB
           + lax.broadcasted_iota(jnp.int32, (ROWB, 1), 0))
    h = jnp.where(row < N, h, 0.0)
    h_ref[...] = h
    hs_ref[...] = h * dinv


def _layer(x, a1, a2, dinv, w1, w2, w3, b):
    return pl.pallas_call(
        _layer_body,
        grid=(NBLK,),
        in_specs=[
            pl.BlockSpec((ROWB, F), lambda i: (i, 0)),
            pl.BlockSpec((NC, ROWB, F), lambda i: (0, i, 0)),
            pl.BlockSpec((NC, ROWB, F), lambda i: (0, i, 0)),
            pl.BlockSpec((ROWB, 16), lambda i: (i, 0)),
            pl.BlockSpec((F, F), lambda i: (0, 0)),
            pl.BlockSpec((F, F), lambda i: (0, 0)),
            pl.BlockSpec((F, F), lambda i: (0, 0)),
            pl.BlockSpec((1, F), lambda i: (0, 0)),
        ],
        out_specs=[
            pl.BlockSpec((ROWB, F), lambda i: (i, 0)),
            pl.BlockSpec((ROWB, F), lambda i: (i, 0)),
        ],
        out_shape=[
            jax.ShapeDtypeStruct((NPAD, F), jnp.float32),
            jax.ShapeDtypeStruct((NPAD, F), jnp.float32),
        ],
    )(x, a1, a2, dinv, w1, w2, w3, b)


def _final_body(x_ref, a1_ref, a2_ref, dinv_ref, w1_ref, w2_ref, w3_ref,
                b_ref, out_ref):
    i = pl.program_id(0)

    @pl.when(i == 0)
    def _():
        out_ref[...] = jnp.zeros_like(out_ref)

    dinv = dinv_ref[:, :1]
    u1 = (a1_ref[0] + a1_ref[1]) * dinv
    u2 = (a2_ref[0] + a2_ref[1]) * dinv
    h = jnp.dot(x_ref[...], w1_ref[...], preferred_element_type=jnp.float32)
    h += jnp.dot(u1, w2_ref[...], preferred_element_type=jnp.float32)
    h += jnp.dot(u2, w3_ref[...], preferred_element_type=jnp.float32)
    row = (i * ROWB + lax.broadcasted_iota(jnp.int32, (ROWB, 1), 0))
    h = jnp.where(row < N, h, 0.0)
    out_ref[...] += jnp.sum(h, axis=0, keepdims=True)

    @pl.when(i == NBLK - 1)
    def _():
        out_ref[...] = out_ref[...] * (1.0 / N) + b_ref[...]


def _final(x, a1, a2, dinv, w1, w2, w3, b):
    return pl.pallas_call(
        _final_body,
        grid=(NBLK,),
        in_specs=[
            pl.BlockSpec((ROWB, F), lambda i: (i, 0)),
            pl.BlockSpec((NC, ROWB, F), lambda i: (0, i, 0)),
            pl.BlockSpec((NC, ROWB, F), lambda i: (0, i, 0)),
            pl.BlockSpec((ROWB, 16), lambda i: (i, 0)),
            pl.BlockSpec((F, N_CLASSES), lambda i: (0, 0)),
            pl.BlockSpec((F, N_CLASSES), lambda i: (0, 0)),
            pl.BlockSpec((F, N_CLASSES), lambda i: (0, 0)),
            pl.BlockSpec((1, N_CLASSES), lambda i: (0, 0)),
        ],
        out_specs=pl.BlockSpec((1, N_CLASSES), lambda i: (0, 0)),
        out_shape=jax.ShapeDtypeStruct((1, N_CLASSES), jnp.float32),
    )(x, a1, a2, dinv, w1, w2, w3, b)


def kernel(features, edge_index, W0, b0, W1, b1, W2, b2):
    npad_e = EPTP - EPT
    pad_dst = jnp.broadcast_to(
        (N + jnp.arange(NW, dtype=jnp.int32))[:, None], (NW, npad_e))
    src = jnp.concatenate(
        (edge_index[0].reshape(NW, EPT), pad_dst), axis=1
    ).reshape(NW, NBATCH, BATCH)
    dst = jnp.concatenate(
        (edge_index[1].reshape(NW, EPT), pad_dst), axis=1
    ).reshape(NW, NBATCH, BATCH)
    zeros_f = jnp.zeros((NPAD, F), jnp.float32)
    ones_f = jnp.ones((BATCH, F), jnp.float32)
    feats = jnp.pad(features, ((0, NPAD - N), (0, 0)))

    deg = _deg_kernel(dst, zeros_f, ones_f)
    dinv, dinv2, xs = _prep(deg, feats)

    def cheb_weights(w, width):
        wa, wb, wc = w[:width], w[width:2 * width], w[2 * width:]
        return wa - wc, -wb, -2.0 * wc

    x = feats
    for w, b in ((W0, b0), (W1, b1)):
        w1e, w2e, w3e = cheb_weights(w, F)
        a1 = _ppass_kernel(src, dst, xs, zeros_f)
        t1s = _mid(a1, dinv2)
        a2 = _ppass_kernel(src, dst, t1s, zeros_f)
        x, xs = _layer(x, a1, a2, dinv, w1e, w2e, w3e, b.reshape(1, F))

    w1e, w2e, w3e = cheb_weights(W2, F)
    a1 = _ppass_kernel(src, dst, xs, zeros_f)
    t1s = _mid(a1, dinv2)
    a2 = _ppass_kernel(src, dst, t1s, zeros_f)
    return _final(x, a1, a2, dinv, w1e, w2e, w3e, b2.reshape(1, N_CLASSES))


# speedup vs baseline: 2.6481x; 1.1874x over previous
import functools

import jax
import jax.numpy as jnp
from jax import lax
from jax.experimental import pallas as pl
from jax.experimental.pallas import tpu as pltpu
from jax.experimental.pallas import tpu_sc as plsc

N = 10000
NPAD = 10240
E = 320000
F = 128
N_CLASSES = 40

NC = 2
NS = 16
NW = NC * NS
EPT = E // NW
EPTP = 10240
BATCH = 80
NBATCH = EPTP // BATCH
CHUNK = 64
NCHUNK = NBATCH // CHUNK
DEPTH = 2

RPT = NPAD // NS

_MESH = plsc.VectorSubcoreMesh(core_axis_name="c", subcore_axis_name="s")

ROWB = 1024
NBLK = NPAD // ROWB


@functools.partial(
    pl.kernel,
    out_type=jax.ShapeDtypeStruct((NC, NPAD, F), jnp.float32),
    mesh=_MESH,
    scratch_types=[
        pltpu.VMEM((NBATCH, BATCH), jnp.int32),
        pltpu.VMEM((BATCH, F), jnp.float32),
        pltpu.VMEM_SHARED((NPAD, F), jnp.float32),
    ],
)
def _deg_kernel(dst_hbm, zeros_hbm, ones_hbm, out_hbm, dst_v, ones_v, cnt_sh):
    c = lax.axis_index("c")
    s = lax.axis_index("s")
    wid = c * NS + s
    pltpu.sync_copy(zeros_hbm.at[pl.ds(s * RPT, RPT)],
                    cnt_sh.at[pl.ds(s * RPT, RPT)])
    pltpu.sync_copy(ones_hbm, ones_v)
    pltpu.sync_copy(dst_hbm.at[wid], dst_v)
    plsc.subcore_barrier()

    def body(j, carry):
        pltpu.sync_copy(ones_v, cnt_sh.at[dst_v.at[j]], add=True)
        return carry

    lax.fori_loop(0, NBATCH, body, 0)
    plsc.subcore_barrier()
    pltpu.sync_copy(cnt_sh.at[pl.ds(s * RPT, RPT)],
                    out_hbm.at[c, pl.ds(s * RPT, RPT)])


@functools.partial(
    pl.kernel,
    out_type=jax.ShapeDtypeStruct((NC, NPAD, F), jnp.float32),
    mesh=_MESH,
    scratch_types=[
        pltpu.VMEM((CHUNK, BATCH), jnp.int32),
        pltpu.VMEM((CHUNK, BATCH), jnp.int32),
        pltpu.VMEM((DEPTH, BATCH, F), jnp.float32),
        pltpu.VMEM_SHARED((NPAD, F), jnp.float32),
        pltpu.SemaphoreType.DMA,
        pltpu.SemaphoreType.DMA,
        pltpu.SemaphoreType.DMA,
        pltpu.SemaphoreType.DMA,
    ],
)
def _ppass_kernel(src_hbm, dst_hbm, x_hbm, zeros_hbm, out_hbm,
                  src_v, dst_v, rows_v, agg_sh, gsem0, gsem1, ssem0, ssem1):
    gsem = (gsem0, gsem1)
    ssem = (ssem0, ssem1)
    c = lax.axis_index("c")
    s = lax.axis_index("s")
    wid = c * NS + s
    pltpu.sync_copy(zeros_hbm.at[pl.ds(s * RPT, RPT)],
                    agg_sh.at[pl.ds(s * RPT, RPT)])
    plsc.subcore_barrier()

    def gstart(j, k):
        pltpu.make_async_copy(x_hbm.at[src_v.at[j]], rows_v.at[k],
                              gsem[k]).start()

    def gwait(j, k):
        pltpu.make_async_copy(x_hbm.at[src_v.at[j]], rows_v.at[k],
                              gsem[k]).wait()

    def sstart(j, k):
        pltpu.make_async_copy(rows_v.at[k], agg_sh.at[dst_v.at[j]],
                              ssem[k]).start(add=True)

    def swait(j, k):
        pltpu.make_async_copy(rows_v.at[k], agg_sh.at[dst_v.at[j]],
                              ssem[k]).wait()

    for chunk in range(NCHUNK):
        pltpu.sync_copy(src_hbm.at[wid, pl.ds(chunk * CHUNK, CHUNK)], src_v)
        pltpu.sync_copy(dst_hbm.at[wid, pl.ds(chunk * CHUNK, CHUNK)], dst_v)
        gstart(0, 0)

        def body(i, carry):
            j0 = i * DEPTH
            gstart(j0 + 1, 1)
            gwait(j0, 0)
            sstart(j0, 0)
            swait(j0, 0)
            gstart(j0 + 2, 0)
            gwait(j0 + 1, 1)
            sstart(j0 + 1, 1)
            swait(j0 + 1, 1)
            return carry

        lax.fori_loop(0, CHUNK // DEPTH - 1, body, 0)
        j0 = CHUNK - DEPTH
        gstart(j0 + 1, 1)
        for k in range(DEPTH):
            gwait(j0 + k, k)
            sstart(j0 + k, k)
            swait(j0 + k, k)

    plsc.subcore_barrier()
    pltpu.sync_copy(agg_sh.at[pl.ds(s * RPT, RPT)],
                    out_hbm.at[c, pl.ds(s * RPT, RPT)])


def _prep_body(deg_ref, x_ref, dinv_ref, dinv2_ref, xs_ref):
    d = jnp.maximum(deg_ref[0][:, :16] + deg_ref[1][:, :16], 1.0)
    dinv2 = 1.0 / d
    dinv = jnp.sqrt(dinv2)
    dinv_ref[...] = dinv
    dinv2_ref[...] = dinv2
    xs_ref[...] = x_ref[...] * dinv[:, :1]


def _prep(deg, x):
    return pl.pallas_call(
        _prep_body,
        grid=(NBLK,),
        in_specs=[
            pl.BlockSpec((NC, ROWB, F), lambda i: (0, i, 0)),
            pl.BlockSpec((ROWB, F), lambda i: (i, 0)),
        ],
        out_specs=[
            pl.BlockSpec((ROWB, 16), lambda i: (i, 0)),
            pl.BlockSpec((ROWB, 16), lambda i: (i, 0)),
            pl.BlockSpec((ROWB, F), lambda i: (i, 0)),
        ],
        out_shape=[
            jax.ShapeDtypeStruct((NPAD, 16), jnp.float32),
            jax.ShapeDtypeStruct((NPAD, 16), jnp.float32),
            jax.ShapeDtypeStruct((NPAD, F), jnp.float32),
        ],
    )(deg, x)


def _mid_body(a1_ref, dinv2_ref, t1s_ref):
    t1s_ref[...] = (a1_ref[0] + a1_ref[1]) * (-dinv2_ref[:, :1])


def _mid(a1, dinv2):
    return pl.pallas_call(
        _mid_body,
        grid=(NBLK,),
        in_specs=[
            pl.BlockSpec((NC, ROWB, F), lambda i: (0, i, 0)),
            pl.BlockSpec((ROWB, 16), lambda i: (i, 0)),
        ],
        out_specs=pl.BlockSpec((ROWB, F), lambda i: (i, 0)),
        out_shape=jax.ShapeDtypeStruct((NPAD, F), jnp.float32),
    )(a1, dinv2)


def _layer_body(x_ref, a1_ref, a2_ref, dinv_ref, w1_ref, w2_ref, w3_ref,
                b_ref, h_ref, hs_ref):
    dinv = dinv_ref[:, :1]
    u1 = (a1_ref[0] + a1_ref[1]) * dinv
    u2 = (a2_ref[0] + a2_ref[1]) * dinv
    h = jnp.dot(x_ref[...], w1_ref[...], preferred_element_type=jnp.float32)
    h += jnp.dot(u1, w2_ref[...], preferred_element_type=jnp.float32)
    h += jnp.dot(u2, w3_ref[...], preferred_element_type=jnp.float32)
    h += b_ref[...]
    row = (pl.program_id(0) * ROWB
           + lax.broadcasted_iota(jnp.int32, (ROWB, 1), 0))
    h = jnp.where(row < N, h, 0.0)
    h_ref[...] = h
    hs_ref[...] = h * dinv


def _layer(x, a1, a2, dinv, w1, w2, w3, b):
    return pl.pallas_call(
        _layer_body,
        grid=(NBLK,),
        in_specs=[
            pl.BlockSpec((ROWB, F), lambda i: (i, 0)),
            pl.BlockSpec((NC, ROWB, F), lambda i: (0, i, 0)),
            pl.BlockSpec((NC, ROWB, F), lambda i: (0, i, 0)),
            pl.BlockSpec((ROWB, 16), lambda i: (i, 0)),
            pl.BlockSpec((F, F), lambda i: (0, 0)),
            pl.BlockSpec((F, F), lambda i: (0, 0)),
            pl.BlockSpec((F, F), lambda i: (0, 0)),
            pl.BlockSpec((1, F), lambda i: (0, 0)),
        ],
        out_specs=[
            pl.BlockSpec((ROWB, F), lambda i: (i, 0)),
            pl.BlockSpec((ROWB, F), lambda i: (i, 0)),
        ],
        out_shape=[
            jax.ShapeDtypeStruct((NPAD, F), jnp.float32),
            jax.ShapeDtypeStruct((NPAD, F), jnp.float32),
        ],
    )(x, a1, a2, dinv, w1, w2, w3, b)


def _final_body(x_ref, a1_ref, a2_ref, dinv_ref, w1_ref, w2_ref, w3_ref,
                b_ref, out_ref):
    i = pl.program_id(0)

    @pl.when(i == 0)
    def _():
        out_ref[...] = jnp.zeros_like(out_ref)

    dinv = dinv_ref[:, :1]
    u1 = (a1_ref[0] + a1_ref[1]) * dinv
    u2 = (a2_ref[0] + a2_ref[1]) * dinv
    h = jnp.dot(x_ref[...], w1_ref[...], preferred_element_type=jnp.float32)
    h += jnp.dot(u1, w2_ref[...], preferred_element_type=jnp.float32)
    h += jnp.dot(u2, w3_ref[...], preferred_element_type=jnp.float32)
    row = (i * ROWB + lax.broadcasted_iota(jnp.int32, (ROWB, 1), 0))
    h = jnp.where(row < N, h, 0.0)
    out_ref[...] += jnp.sum(h, axis=0, keepdims=True)

    @pl.when(i == NBLK - 1)
    def _():
        out_ref[...] = out_ref[...] * (1.0 / N) + b_ref[...]


def _final(x, a1, a2, dinv, w1, w2, w3, b):
    return pl.pallas_call(
        _final_body,
        grid=(NBLK,),
        in_specs=[
            pl.BlockSpec((ROWB, F), lambda i: (i, 0)),
            pl.BlockSpec((NC, ROWB, F), lambda i: (0, i, 0)),
            pl.BlockSpec((NC, ROWB, F), lambda i: (0, i, 0)),
            pl.BlockSpec((ROWB, 16), lambda i: (i, 0)),
            pl.BlockSpec((F, N_CLASSES), lambda i: (0, 0)),
            pl.BlockSpec((F, N_CLASSES), lambda i: (0, 0)),
            pl.BlockSpec((F, N_CLASSES), lambda i: (0, 0)),
            pl.BlockSpec((1, N_CLASSES), lambda i: (0, 0)),
        ],
        out_specs=pl.BlockSpec((1, N_CLASSES), lambda i: (0, 0)),
        out_shape=jax.ShapeDtypeStruct((1, N_CLASSES), jnp.float32),
    )(x, a1, a2, dinv, w1, w2, w3, b)


def kernel(features, edge_index, W0, b0, W1, b1, W2, b2):
    npad_e = EPTP - EPT
    pad_dst = jnp.broadcast_to(
        (N + jnp.arange(NW, dtype=jnp.int32))[:, None], (NW, npad_e))
    src = jnp.concatenate(
        (edge_index[0].reshape(NW, EPT), pad_dst), axis=1
    ).reshape(NW, NBATCH, BATCH)
    dst = jnp.concatenate(
        (edge_index[1].reshape(NW, EPT), pad_dst), axis=1
    ).reshape(NW, NBATCH, BATCH)
    zeros_f = jnp.zeros((NPAD, F), jnp.float32)
    ones_f = jnp.ones((BATCH, F), jnp.float32)
    feats = jnp.pad(features, ((0, NPAD - N), (0, 0)))

    deg = _deg_kernel(dst, zeros_f, ones_f)
    dinv, dinv2, xs = _prep(deg, feats)

    def cheb_weights(w, width):
        wa, wb, wc = w[:width], w[width:2 * width], w[2 * width:]
        return wa - wc, -wb, -2.0 * wc

    x = feats
    for w, b in ((W0, b0), (W1, b1)):
        w1e, w2e, w3e = cheb_weights(w, F)
        a1 = _ppass_kernel(src, dst, xs, zeros_f)
        t1s = _mid(a1, dinv2)
        a2 = _ppass_kernel(src, dst, t1s, zeros_f)
        x, xs = _layer(x, a1, a2, dinv, w1e, w2e, w3e, b.reshape(1, F))

    w1e, w2e, w3e = cheb_weights(W2, F)
    a1 = _ppass_kernel(src, dst, xs, zeros_f)
    t1s = _mid(a1, dinv2)
    a2 = _ppass_kernel(src, dst, t1s, zeros_f)
    return _final(x, a1, a2, dinv, w1e, w2e, w3e, b2.reshape(1, N_CLASSES))


# speedup vs baseline: 2.7084x; 1.0228x over previous
import functools

import jax
import jax.numpy as jnp
from jax import lax
from jax.experimental import pallas as pl
from jax.experimental.pallas import tpu as pltpu
from jax.experimental.pallas import tpu_sc as plsc

N = 10000
NPAD = 10240
E = 320000
F = 128
N_CLASSES = 40

NC = 2
NS = 16
NW = NC * NS
EPT = E // NW
EPTP = 10240
BATCH = 80
NBATCH = EPTP // BATCH
CHUNK = 16
NCHUNK = NBATCH // CHUNK
DEPTH = 4

RPT = NPAD // NS

_MESH = plsc.VectorSubcoreMesh(core_axis_name="c", subcore_axis_name="s")

ROWB = 1024
NBLK = NPAD // ROWB


@functools.partial(
    pl.kernel,
    out_type=jax.ShapeDtypeStruct((NC, NPAD, F), jnp.float32),
    mesh=_MESH,
    scratch_types=[
        pltpu.VMEM((NBATCH, BATCH), jnp.int32),
        pltpu.VMEM((BATCH, F), jnp.float32),
        pltpu.VMEM_SHARED((NPAD, F), jnp.float32),
    ],
)
def _deg_kernel(dst_hbm, zeros_hbm, ones_hbm, out_hbm, dst_v, ones_v, cnt_sh):
    c = lax.axis_index("c")
    s = lax.axis_index("s")
    wid = c * NS + s
    pltpu.sync_copy(zeros_hbm.at[pl.ds(s * RPT, RPT)],
                    cnt_sh.at[pl.ds(s * RPT, RPT)])
    pltpu.sync_copy(ones_hbm, ones_v)
    pltpu.sync_copy(dst_hbm.at[wid], dst_v)
    plsc.subcore_barrier()

    def body(j, carry):
        pltpu.sync_copy(ones_v, cnt_sh.at[dst_v.at[j]], add=True)
        return carry

    lax.fori_loop(0, NBATCH, body, 0)
    plsc.subcore_barrier()
    pltpu.sync_copy(cnt_sh.at[pl.ds(s * RPT, RPT)],
                    out_hbm.at[c, pl.ds(s * RPT, RPT)])


@functools.partial(
    pl.kernel,
    out_type=jax.ShapeDtypeStruct((NC, NPAD, F), jnp.float32),
    mesh=_MESH,
    scratch_types=[
        pltpu.VMEM((CHUNK, BATCH), jnp.int32),
        pltpu.VMEM((CHUNK, BATCH), jnp.int32),
        pltpu.VMEM((DEPTH, BATCH, F), jnp.float32),
        pltpu.VMEM_SHARED((NPAD, F), jnp.float32),
        pltpu.SemaphoreType.DMA,
        pltpu.SemaphoreType.DMA,
        pltpu.SemaphoreType.DMA,
        pltpu.SemaphoreType.DMA,
        pltpu.SemaphoreType.DMA,
        pltpu.SemaphoreType.DMA,
        pltpu.SemaphoreType.DMA,
        pltpu.SemaphoreType.DMA,
    ],
)
def _ppass_kernel(src_hbm, dst_hbm, x_hbm, zeros_hbm, out_hbm,
                  src_v, dst_v, rows_v, agg_sh,
                  gsem0, gsem1, gsem2, gsem3, ssem0, ssem1, ssem2, ssem3):
    gsem = (gsem0, gsem1, gsem2, gsem3)
    ssem = (ssem0, ssem1, ssem2, ssem3)
    c = lax.axis_index("c")
    s = lax.axis_index("s")
    wid = c * NS + s
    pltpu.sync_copy(zeros_hbm.at[pl.ds(s * RPT, RPT)],
                    agg_sh.at[pl.ds(s * RPT, RPT)])
    plsc.subcore_barrier()

    def gstart(j, k):
        pltpu.make_async_copy(x_hbm.at[src_v.at[j]], rows_v.at[k],
                              gsem[k]).start()

    def gwait(j, k):
        pltpu.make_async_copy(x_hbm.at[src_v.at[j]], rows_v.at[k],
                              gsem[k]).wait()

    def sstart(j, k):
        pltpu.make_async_copy(rows_v.at[k], agg_sh.at[dst_v.at[j]],
                              ssem[k]).start(add=True)

    def swait(j, k):
        pltpu.make_async_copy(rows_v.at[k], agg_sh.at[dst_v.at[j]],
                              ssem[k]).wait()

    for chunk in range(NCHUNK):
        pltpu.sync_copy(src_hbm.at[wid, pl.ds(chunk * CHUNK, CHUNK)], src_v)
        pltpu.sync_copy(dst_hbm.at[wid, pl.ds(chunk * CHUNK, CHUNK)], dst_v)
        for k in range(DEPTH - 1):
            gstart(k, k)

        def body(i, carry):
            j0 = i * DEPTH
            for k in range(DEPTH):
                gstart(j0 + k + DEPTH - 1, (k + DEPTH - 1) % DEPTH)
                gwait(j0 + k, k)
                sstart(j0 + k, k)
                swait(j0 + k, k)
            return carry

        lax.fori_loop(0, CHUNK // DEPTH - 1, body, 0)
        j0 = CHUNK - DEPTH
        gstart(CHUNK - 1, DEPTH - 1)
        for k in range(DEPTH):
            gwait(j0 + k, k)
            sstart(j0 + k, k)
            swait(j0 + k, k)

    plsc.subcore_barrier()
    pltpu.sync_copy(agg_sh.at[pl.ds(s * RPT, RPT)],
                    out_hbm.at[c, pl.ds(s * RPT, RPT)])


def _prep_body(deg_ref, x_ref, dinv_ref, dinv2_ref, xs_ref):
    d = jnp.maximum(deg_ref[0][:, :16] + deg_ref[1][:, :16], 1.0)
    dinv2 = 1.0 / d
    dinv = jnp.sqrt(dinv2)
    dinv_ref[...] = dinv
    dinv2_ref[...] = dinv2
    xs_ref[...] = x_ref[...] * dinv[:, :1]


def _prep(deg, x):
    return pl.pallas_call(
        _prep_body,
        grid=(NBLK,),
        in_specs=[
            pl.BlockSpec((NC, ROWB, F), lambda i: (0, i, 0)),
            pl.BlockSpec((ROWB, F), lambda i: (i, 0)),
        ],
        out_specs=[
            pl.BlockSpec((ROWB, 16), lambda i: (i, 0)),
            pl.BlockSpec((ROWB, 16), lambda i: (i, 0)),
            pl.BlockSpec((ROWB, F), lambda i: (i, 0)),
        ],
        out_shape=[
            jax.ShapeDtypeStruct((NPAD, 16), jnp.float32),
            jax.ShapeDtypeStruct((NPAD, 16), jnp.float32),
            jax.ShapeDtypeStruct((NPAD, F), jnp.float32),
        ],
    )(deg, x)


def _mid_body(a1_ref, dinv2_ref, t1s_ref):
    t1s_ref[...] = (a1_ref[0] + a1_ref[1]) * (-dinv2_ref[:, :1])


def _mid(a1, dinv2):
    return pl.pallas_call(
        _mid_body,
        grid=(NBLK,),
        in_specs=[
            pl.BlockSpec((NC, ROWB, F), lambda i: (0, i, 0)),
            pl.BlockSpec((ROWB, 16), lambda i: (i, 0)),
        ],
        out_specs=pl.BlockSpec((ROWB, F), lambda i: (i, 0)),
        out_shape=jax.ShapeDtypeStruct((NPAD, F), jnp.float32),
    )(a1, dinv2)


def _layer_body(x_ref, a1_ref, a2_ref, dinv_ref, w1_ref, w2_ref, w3_ref,
                b_ref, h_ref, hs_ref):
    dinv = dinv_ref[:, :1]
    u1 = (a1_ref[0] + a1_ref[1]) * dinv
    u2 = (a2_ref[0] + a2_ref[1]) * dinv
    h = jnp.dot(x_ref[...], w1_ref[...], preferred_element_type=jnp.float32)
    h += jnp.dot(u1, w2_ref[...], preferred_element_type=jnp.float32)
    h += jnp.dot(u2, w3_ref[...], preferred_element_type=jnp.float32)
    h += b_ref[...]
    row = (pl.program_id(0) * ROWB
           + lax.broadcasted_iota(jnp.int32, (ROWB, 1), 0))
    h = jnp.where(row < N, h, 0.0)
    h_ref[...] = h
    hs_ref[...] = h * dinv


def _layer(x, a1, a2, dinv, w1, w2, w3, b):
    return pl.pallas_call(
        _layer_body,
        grid=(NBLK,),
        in_specs=[
            pl.BlockSpec((ROWB, F), lambda i: (i, 0)),
            pl.BlockSpec((NC, ROWB, F), lambda i: (0, i, 0)),
            pl.BlockSpec((NC, ROWB, F), lambda i: (0, i, 0)),
            pl.BlockSpec((ROWB, 16), lambda i: (i, 0)),
            pl.BlockSpec((F, F), lambda i: (0, 0)),
            pl.BlockSpec((F, F), lambda i: (0, 0)),
            pl.BlockSpec((F, F), lambda i: (0, 0)),
            pl.BlockSpec((1, F), lambda i: (0, 0)),
        ],
        out_specs=[
            pl.BlockSpec((ROWB, F), lambda i: (i, 0)),
            pl.BlockSpec((ROWB, F), lambda i: (i, 0)),
        ],
        out_shape=[
            jax.ShapeDtypeStruct((NPAD, F), jnp.float32),
            jax.ShapeDtypeStruct((NPAD, F), jnp.float32),
        ],
    )(x, a1, a2, dinv, w1, w2, w3, b)


def _final_body(x_ref, a1_ref, a2_ref, dinv_ref, w1_ref, w2_ref, w3_ref,
                b_ref, out_ref):
    i = pl.program_id(0)

    @pl.when(i == 0)
    def _():
        out_ref[...] = jnp.zeros_like(out_ref)

    dinv = dinv_ref[:, :1]
    u1 = (a1_ref[0] + a1_ref[1]) * dinv
    u2 = (a2_ref[0] + a2_ref[1]) * dinv
    h = jnp.dot(x_ref[...], w1_ref[...], preferred_element_type=jnp.float32)
    h += jnp.dot(u1, w2_ref[...], preferred_element_type=jnp.float32)
    h += jnp.dot(u2, w3_ref[...], preferred_element_type=jnp.float32)
    row = (i * ROWB + lax.broadcasted_iota(jnp.int32, (ROWB, 1), 0))
    h = jnp.where(row < N, h, 0.0)
    out_ref[...] += jnp.sum(h, axis=0, keepdims=True)

    @pl.when(i == NBLK - 1)
    def _():
        out_ref[...] = out_ref[...] * (1.0 / N) + b_ref[...]


def _final(x, a1, a2, dinv, w1, w2, w3, b):
    return pl.pallas_call(
        _final_body,
        grid=(NBLK,),
        in_specs=[
            pl.BlockSpec((ROWB, F), lambda i: (i, 0)),
            pl.BlockSpec((NC, ROWB, F), lambda i: (0, i, 0)),
            pl.BlockSpec((NC, ROWB, F), lambda i: (0, i, 0)),
            pl.BlockSpec((ROWB, 16), lambda i: (i, 0)),
            pl.BlockSpec((F, N_CLASSES), lambda i: (0, 0)),
            pl.BlockSpec((F, N_CLASSES), lambda i: (0, 0)),
            pl.BlockSpec((F, N_CLASSES), lambda i: (0, 0)),
            pl.BlockSpec((1, N_CLASSES), lambda i: (0, 0)),
        ],
        out_specs=pl.BlockSpec((1, N_CLASSES), lambda i: (0, 0)),
        out_shape=jax.ShapeDtypeStruct((1, N_CLASSES), jnp.float32),
    )(x, a1, a2, dinv, w1, w2, w3, b)


def kernel(features, edge_index, W0, b0, W1, b1, W2, b2):
    npad_e = EPTP - EPT
    pad_dst = jnp.broadcast_to(
        (N + jnp.arange(NW, dtype=jnp.int32))[:, None], (NW, npad_e))
    src = jnp.concatenate(
        (edge_index[0].reshape(NW, EPT), pad_dst), axis=1
    ).reshape(NW, NBATCH, BATCH)
    dst = jnp.concatenate(
        (edge_index[1].reshape(NW, EPT), pad_dst), axis=1
    ).reshape(NW, NBATCH, BATCH)
    zeros_f = jnp.zeros((NPAD, F), jnp.float32)
    ones_f = jnp.ones((BATCH, F), jnp.float32)
    feats = jnp.pad(features, ((0, NPAD - N), (0, 0)))

    deg = _deg_kernel(dst, zeros_f, ones_f)
    dinv, dinv2, xs = _prep(deg, feats)

    def cheb_weights(w, width):
        wa, wb, wc = w[:width], w[width:2 * width], w[2 * width:]
        return wa - wc, -wb, -2.0 * wc

    x = feats
    for w, b in ((W0, b0), (W1, b1)):
        w1e, w2e, w3e = cheb_weights(w, F)
        a1 = _ppass_kernel(src, dst, xs, zeros_f)
        t1s = _mid(a1, dinv2)
        a2 = _ppass_kernel(src, dst, t1s, zeros_f)
        x, xs = _layer(x, a1, a2, dinv, w1e, w2e, w3e, b.reshape(1, F))

    w1e, w2e, w3e = cheb_weights(W2, F)
    a1 = _ppass_kernel(src, dst, xs, zeros_f)
    t1s = _mid(a1, dinv2)
    a2 = _ppass_kernel(src, dst, t1s, zeros_f)
    return _final(x, a1, a2, dinv, w1e, w2e, w3e, b2.reshape(1, N_CLASSES))


# speedup vs baseline: 2.9186x; 1.0776x over previous
import functools

import jax
import jax.numpy as jnp
from jax import lax
from jax.experimental import pallas as pl
from jax.experimental.pallas import tpu as pltpu
from jax.experimental.pallas import tpu_sc as plsc

N = 10000
NPAD = 10112
E = 320000
F = 128
N_CLASSES = 40

NC = 2
NS = 16
NW = NC * NS
EPT = E // NW
EPTP = 10240
BATCH = 80
NBATCH = EPTP // BATCH
CHUNK = 32
NCHUNK = NBATCH // CHUNK
DEPTH = 4

RPT = NPAD // NS

_MESH = plsc.VectorSubcoreMesh(core_axis_name="c", subcore_axis_name="s")

ROWB = 632
NBLK = NPAD // ROWB


@functools.partial(
    pl.kernel,
    out_type=jax.ShapeDtypeStruct((NC, NPAD, F), jnp.float32),
    mesh=_MESH,
    scratch_types=[
        pltpu.VMEM((NBATCH, BATCH), jnp.int32),
        pltpu.VMEM((BATCH, F), jnp.float32),
        pltpu.VMEM_SHARED((NPAD, F), jnp.float32),
    ],
)
def _deg_kernel(dst_hbm, zeros_hbm, ones_hbm, out_hbm, dst_v, ones_v, cnt_sh):
    c = lax.axis_index("c")
    s = lax.axis_index("s")
    wid = c * NS + s
    pltpu.sync_copy(zeros_hbm.at[pl.ds(s * RPT, RPT)],
                    cnt_sh.at[pl.ds(s * RPT, RPT)])
    pltpu.sync_copy(ones_hbm, ones_v)
    pltpu.sync_copy(dst_hbm.at[wid], dst_v)
    plsc.subcore_barrier()

    def body(j, carry):
        pltpu.sync_copy(ones_v, cnt_sh.at[dst_v.at[j]], add=True)
        return carry

    lax.fori_loop(0, NBATCH, body, 0)
    plsc.subcore_barrier()
    pltpu.sync_copy(cnt_sh.at[pl.ds(s * RPT, RPT)],
                    out_hbm.at[c, pl.ds(s * RPT, RPT)])


@functools.partial(
    pl.kernel,
    out_type=jax.ShapeDtypeStruct((NC, NPAD, F), jnp.float32),
    mesh=_MESH,
    scratch_types=[
        pltpu.VMEM((CHUNK, BATCH), jnp.int32),
        pltpu.VMEM((CHUNK, BATCH), jnp.int32),
        pltpu.VMEM((DEPTH, BATCH, F), jnp.float32),
        pltpu.VMEM_SHARED((NPAD, F), jnp.float32),
        pltpu.SemaphoreType.DMA,
        pltpu.SemaphoreType.DMA,
        pltpu.SemaphoreType.DMA,
        pltpu.SemaphoreType.DMA,
        pltpu.SemaphoreType.DMA,
        pltpu.SemaphoreType.DMA,
        pltpu.SemaphoreType.DMA,
        pltpu.SemaphoreType.DMA,
    ],
)
def _ppass_kernel(src_hbm, dst_hbm, x_hbm, zeros_hbm, out_hbm,
                  src_v, dst_v, rows_v, agg_sh,
                  gsem0, gsem1, gsem2, gsem3, ssem0, ssem1, ssem2, ssem3):
    gsem = (gsem0, gsem1, gsem2, gsem3)
    ssem = (ssem0, ssem1, ssem2, ssem3)
    c = lax.axis_index("c")
    s = lax.axis_index("s")
    wid = c * NS + s
    pltpu.sync_copy(zeros_hbm.at[pl.ds(s * RPT, RPT)],
                    agg_sh.at[pl.ds(s * RPT, RPT)])
    plsc.subcore_barrier()

    def gstart(j, k):
        pltpu.make_async_copy(x_hbm.at[src_v.at[j]], rows_v.at[k],
                              gsem[k]).start()

    def gwait(j, k):
        pltpu.make_async_copy(x_hbm.at[src_v.at[j]], rows_v.at[k],
                              gsem[k]).wait()

    def sstart(j, k):
        pltpu.make_async_copy(rows_v.at[k], agg_sh.at[dst_v.at[j]],
                              ssem[k]).start(add=True)

    def swait(j, k):
        pltpu.make_async_copy(rows_v.at[k], agg_sh.at[dst_v.at[j]],
                              ssem[k]).wait()

    for chunk in range(NCHUNK):
        pltpu.sync_copy(src_hbm.at[wid, pl.ds(chunk * CHUNK, CHUNK)], src_v)
        pltpu.sync_copy(dst_hbm.at[wid, pl.ds(chunk * CHUNK, CHUNK)], dst_v)
        for k in range(DEPTH - 1):
            gstart(k, k)

        def body(i, carry):
            j0 = i * DEPTH
            for k in range(DEPTH):
                gstart(j0 + k + DEPTH - 1, (k + DEPTH - 1) % DEPTH)
                gwait(j0 + k, k)
                sstart(j0 + k, k)
                swait(j0 + k, k)
            return carry

        lax.fori_loop(0, CHUNK // DEPTH - 1, body, 0)
        j0 = CHUNK - DEPTH
        gstart(CHUNK - 1, DEPTH - 1)
        for k in range(DEPTH):
            gwait(j0 + k, k)
            sstart(j0 + k, k)
            swait(j0 + k, k)

    plsc.subcore_barrier()
    pltpu.sync_copy(agg_sh.at[pl.ds(s * RPT, RPT)],
                    out_hbm.at[c, pl.ds(s * RPT, RPT)])


def _prep_body(deg_ref, x_ref, dinv_ref, dinv2_ref, xs_ref):
    d = jnp.maximum(deg_ref[0][:, :16] + deg_ref[1][:, :16], 1.0)
    dinv2 = 1.0 / d
    dinv = jnp.sqrt(dinv2)
    dinv_ref[...] = dinv
    dinv2_ref[...] = dinv2
    xs_ref[...] = x_ref[...] * dinv[:, :1]


def _prep(deg, x):
    return pl.pallas_call(
        _prep_body,
        grid=(NBLK,),
        in_specs=[
            pl.BlockSpec((NC, ROWB, F), lambda i: (0, i, 0)),
            pl.BlockSpec((ROWB, F), lambda i: (i, 0)),
        ],
        out_specs=[
            pl.BlockSpec((ROWB, 16), lambda i: (i, 0)),
            pl.BlockSpec((ROWB, 16), lambda i: (i, 0)),
            pl.BlockSpec((ROWB, F), lambda i: (i, 0)),
        ],
        out_shape=[
            jax.ShapeDtypeStruct((NPAD, 16), jnp.float32),
            jax.ShapeDtypeStruct((NPAD, 16), jnp.float32),
            jax.ShapeDtypeStruct((NPAD, F), jnp.float32),
        ],
    )(deg, x)


def _mid_body(a1_ref, dinv2_ref, t1s_ref):
    t1s_ref[...] = (a1_ref[0] + a1_ref[1]) * (-dinv2_ref[:, :1])


def _mid(a1, dinv2):
    return pl.pallas_call(
        _mid_body,
        grid=(NBLK,),
        in_specs=[
            pl.BlockSpec((NC, ROWB, F), lambda i: (0, i, 0)),
            pl.BlockSpec((ROWB, 16), lambda i: (i, 0)),
        ],
        out_specs=pl.BlockSpec((ROWB, F), lambda i: (i, 0)),
        out_shape=jax.ShapeDtypeStruct((NPAD, F), jnp.float32),
    )(a1, dinv2)


def _layer_body(x_ref, a1_ref, a2_ref, dinv_ref, w1_ref, w2_ref, w3_ref,
                b_ref, h_ref, hs_ref):
    dinv = dinv_ref[:, :1]
    u1 = (a1_ref[0] + a1_ref[1]) * dinv
    u2 = (a2_ref[0] + a2_ref[1]) * dinv
    h = jnp.dot(x_ref[...], w1_ref[...], preferred_element_type=jnp.float32)
    h += jnp.dot(u1, w2_ref[...], preferred_element_type=jnp.float32)
    h += jnp.dot(u2, w3_ref[...], preferred_element_type=jnp.float32)
    h += b_ref[...]
    row = (pl.program_id(0) * ROWB
           + lax.broadcasted_iota(jnp.int32, (ROWB, 1), 0))
    h = jnp.where(row < N, h, 0.0)
    h_ref[...] = h
    hs_ref[...] = h * dinv


def _layer(x, a1, a2, dinv, w1, w2, w3, b):
    return pl.pallas_call(
        _layer_body,
        grid=(NBLK,),
        in_specs=[
            pl.BlockSpec((ROWB, F), lambda i: (i, 0)),
            pl.BlockSpec((NC, ROWB, F), lambda i: (0, i, 0)),
            pl.BlockSpec((NC, ROWB, F), lambda i: (0, i, 0)),
            pl.BlockSpec((ROWB, 16), lambda i: (i, 0)),
            pl.BlockSpec((F, F), lambda i: (0, 0)),
            pl.BlockSpec((F, F), lambda i: (0, 0)),
            pl.BlockSpec((F, F), lambda i: (0, 0)),
            pl.BlockSpec((1, F), lambda i: (0, 0)),
        ],
        out_specs=[
            pl.BlockSpec((ROWB, F), lambda i: (i, 0)),
            pl.BlockSpec((ROWB, F), lambda i: (i, 0)),
        ],
        out_shape=[
            jax.ShapeDtypeStruct((NPAD, F), jnp.float32),
            jax.ShapeDtypeStruct((NPAD, F), jnp.float32),
        ],
    )(x, a1, a2, dinv, w1, w2, w3, b)


def _final_body(x_ref, a1_ref, a2_ref, dinv_ref, w1_ref, w2_ref, w3_ref,
                b_ref, out_ref):
    i = pl.program_id(0)

    @pl.when(i == 0)
    def _():
        out_ref[...] = jnp.zeros_like(out_ref)

    dinv = dinv_ref[:, :1]
    u1 = (a1_ref[0] + a1_ref[1]) * dinv
    u2 = (a2_ref[0] + a2_ref[1]) * dinv
    h = jnp.dot(x_ref[...], w1_ref[...], preferred_element_type=jnp.float32)
    h += jnp.dot(u1, w2_ref[...], preferred_element_type=jnp.float32)
    h += jnp.dot(u2, w3_ref[...], preferred_element_type=jnp.float32)
    row = (i * ROWB + lax.broadcasted_iota(jnp.int32, (ROWB, 1), 0))
    h = jnp.where(row < N, h, 0.0)
    out_ref[...] += jnp.sum(h, axis=0, keepdims=True)

    @pl.when(i == NBLK - 1)
    def _():
        out_ref[...] = out_ref[...] * (1.0 / N) + b_ref[...]


def _final(x, a1, a2, dinv, w1, w2, w3, b):
    return pl.pallas_call(
        _final_body,
        grid=(NBLK,),
        in_specs=[
            pl.BlockSpec((ROWB, F), lambda i: (i, 0)),
            pl.BlockSpec((NC, ROWB, F), lambda i: (0, i, 0)),
            pl.BlockSpec((NC, ROWB, F), lambda i: (0, i, 0)),
            pl.BlockSpec((ROWB, 16), lambda i: (i, 0)),
            pl.BlockSpec((F, N_CLASSES), lambda i: (0, 0)),
            pl.BlockSpec((F, N_CLASSES), lambda i: (0, 0)),
            pl.BlockSpec((F, N_CLASSES), lambda i: (0, 0)),
            pl.BlockSpec((1, N_CLASSES), lambda i: (0, 0)),
        ],
        out_specs=pl.BlockSpec((1, N_CLASSES), lambda i: (0, 0)),
        out_shape=jax.ShapeDtypeStruct((1, N_CLASSES), jnp.float32),
    )(x, a1, a2, dinv, w1, w2, w3, b)


def kernel(features, edge_index, W0, b0, W1, b1, W2, b2):
    npad_e = EPTP - EPT
    pad_dst = jnp.broadcast_to(
        (N + jnp.arange(NW, dtype=jnp.int32))[:, None], (NW, npad_e))
    src = jnp.concatenate(
        (edge_index[0].reshape(NW, EPT), pad_dst), axis=1
    ).reshape(NW, NBATCH, BATCH)
    dst = jnp.concatenate(
        (edge_index[1].reshape(NW, EPT), pad_dst), axis=1
    ).reshape(NW, NBATCH, BATCH)
    zeros_f = jnp.zeros((NPAD, F), jnp.float32)
    ones_f = jnp.ones((BATCH, F), jnp.float32)
    feats = jnp.pad(features, ((0, NPAD - N), (0, 0)))

    deg = _deg_kernel(dst, zeros_f, ones_f)
    dinv, dinv2, xs = _prep(deg, feats)

    def cheb_weights(w, width):
        wa, wb, wc = w[:width], w[width:2 * width], w[2 * width:]
        return wa - wc, -wb, -2.0 * wc

    x = feats
    for w, b in ((W0, b0), (W1, b1)):
        w1e, w2e, w3e = cheb_weights(w, F)
        a1 = _ppass_kernel(src, dst, xs, zeros_f)
        t1s = _mid(a1, dinv2)
        a2 = _ppass_kernel(src, dst, t1s, zeros_f)
        x, xs = _layer(x, a1, a2, dinv, w1e, w2e, w3e, b.reshape(1, F))

    w1e, w2e, w3e = cheb_weights(W2, F)
    a1 = _ppass_kernel(src, dst, xs, zeros_f)
    t1s = _mid(a1, dinv2)
    a2 = _ppass_kernel(src, dst, t1s, zeros_f)
    return _final(x, a1, a2, dinv, w1e, w2e, w3e, b2.reshape(1, N_CLASSES))
